# Initial kernel scaffold; baseline (speedup 1.0000x reference)
import functools

import jax
import jax.numpy as jnp
from jax import lax
from jax.experimental import pallas as pl
from jax.experimental.pallas import tpu as pltpu
from jax.experimental.pallas import tpu_sc as plsc

NC = 2
NS = 16
NW = NC * NS

GRP = 128
CHUNK = 13 * GRP


def _make_gather(B, V, D):
    assert B % NW == 0
    b_per_w = B // NW
    assert b_per_w % CHUNK == 0
    n_chunks = b_per_w // CHUNK
    mesh = plsc.VectorSubcoreMesh(
        core_axis_name="c", subcore_axis_name="s", num_cores=NC, num_subcores=NS
    )

    @functools.partial(
        pl.kernel,
        out_type=jax.ShapeDtypeStruct((B, D), jnp.float32),
        mesh=mesh,
        scratch_types=[
            pltpu.VMEM((CHUNK // GRP, GRP), jnp.int32),
            pltpu.VMEM((CHUNK, D), jnp.float32),
            pltpu.SemaphoreType.DMA,
        ],
    )
    def gather_kernel(idx_hbm, table_hbm, out_hbm, idx_v, rows_v, sem):
        wid = lax.axis_index("s") * NC + lax.axis_index("c")
        base = wid * b_per_w

        def body(i, carry):
            off = pl.multiple_of(base + i * CHUNK, CHUNK)
            pltpu.sync_copy(idx_hbm.at[pl.ds(off, CHUNK)], idx_v)
            copies = []
            for j in range(CHUNK // GRP):
                copies.append(
                    pltpu.async_copy(
                        table_hbm.at[idx_v.at[j]],
                        rows_v.at[pl.ds(j * GRP, GRP)],
                        sem,
                    )
                )
            for c in copies:
                c.wait()
            pltpu.sync_copy(rows_v, out_hbm.at[pl.ds(off, CHUNK)])
            return carry

        lax.fori_loop(0, n_chunks, body, 0)

    return gather_kernel


def _norm_body(x_ref, o_ref):
    x = x_ref[...]
    s = jnp.sum(x * x, axis=-1, keepdims=True)
    o_ref[...] = x * lax.rsqrt(jnp.maximum(s, 1e-24))


def _normalize(rows):
    B, D = rows.shape
    BLK = 4096
    assert B % BLK == 0
    return pl.pallas_call(
        _norm_body,
        grid=(B // BLK,),
        in_specs=[pl.BlockSpec((BLK, D), lambda i: (i, 0))],
        out_specs=pl.BlockSpec((BLK, D), lambda i: (i, 0)),
        out_shape=jax.ShapeDtypeStruct((B, D), jnp.float32),
    )(rows)


def kernel(input, W):
    batch, fields = input.shape
    V, D = W.shape
    B = batch * fields
    idx = input.reshape(B).astype(jnp.int32)
    rows = _make_gather(B, V, D)(idx, W)
    out = _normalize(rows)
    return out.reshape(batch, fields, D)


# speedup vs baseline: 1.1294x; 1.1294x over previous
import functools

import jax
import jax.numpy as jnp
from jax import lax
from jax.experimental import pallas as pl
from jax.experimental.pallas import tpu as pltpu
from jax.experimental.pallas import tpu_sc as plsc

NC = 2
NS = 16
NW = NC * NS

GRP = 128
CHUNK = 8 * GRP


def _make_gather(B, V, D):
    assert B % NW == 0
    b_per_w = B // NW
    assert b_per_w % CHUNK == 0
    n_chunks = b_per_w // CHUNK
    mesh = plsc.VectorSubcoreMesh(
        core_axis_name="c", subcore_axis_name="s", num_cores=NC, num_subcores=NS
    )

    @functools.partial(
        pl.kernel,
        out_type=jax.ShapeDtypeStruct((B, D), jnp.float32),
        mesh=mesh,
        scratch_types=[
            pltpu.VMEM((CHUNK // GRP, GRP), jnp.int32),
            pltpu.VMEM((CHUNK, D), jnp.float32),
            pltpu.SemaphoreType.DMA,
        ],
        compiler_params=pltpu.CompilerParams(use_tc_tiling_on_sc=False),
    )
    def gather_kernel(idx_hbm, table_hbm, out_hbm, idx_v, rows_v, sem):
        wid = lax.axis_index("s") * NC + lax.axis_index("c")
        base = wid * b_per_w

        def body(i, carry):
            off = pl.multiple_of(base + i * CHUNK, CHUNK)
            row_off = pl.multiple_of(off // GRP, CHUNK // GRP)
            pltpu.sync_copy(idx_hbm.at[pl.ds(row_off, CHUNK // GRP)], idx_v)
            copies = []
            for j in range(CHUNK // GRP):
                copies.append(
                    pltpu.async_copy(
                        table_hbm.at[idx_v.at[j]],
                        rows_v.at[pl.ds(j * GRP, GRP)],
                        sem,
                    )
                )
            for c in copies:
                c.wait()
            pltpu.sync_copy(rows_v, out_hbm.at[pl.ds(off, CHUNK)])
            return carry

        lax.fori_loop(0, n_chunks, body, 0)

    return gather_kernel


def _norm_body(x_ref, o_ref):
    x = x_ref[...]
    s = jnp.sum(x * x, axis=-1, keepdims=True)
    o_ref[...] = x * lax.rsqrt(jnp.maximum(s, 1e-24))


def _normalize(rows):
    B, D = rows.shape
    BLK = 4096
    assert B % BLK == 0
    return pl.pallas_call(
        _norm_body,
        grid=(B // BLK,),
        in_specs=[pl.BlockSpec((BLK, D), lambda i: (i, 0))],
        out_specs=pl.BlockSpec((BLK, D), lambda i: (i, 0)),
        out_shape=jax.ShapeDtypeStruct((B, D), jnp.float32),
    )(rows)


def kernel(input, W):
    batch, fields = input.shape
    V, D = W.shape
    B = batch * fields
    idx = input.reshape(B // GRP, GRP).astype(jnp.int32)
    rows = _make_gather(B, V, D)(idx, W)
    out = _normalize(rows)
    return out.reshape(batch, fields, D)


# speedup vs baseline: 1.1456x; 1.0144x over previous
import functools

import jax
import jax.numpy as jnp
from jax import lax
from jax.experimental import pallas as pl
from jax.experimental.pallas import tpu as pltpu
from jax.experimental.pallas import tpu_sc as plsc

NC = 2
NS = 16
NW = NC * NS

GRP = 128
ROWS_PER_GROUP = 16


def _rsqrt(x):
    i = plsc.bitcast(x, jnp.int32)
    i = 0x5F3759DF - lax.shift_right_logical(i, 1)
    y = plsc.bitcast(i, jnp.float32)
    xh = x * 0.5
    for _ in range(3):
        y = y * (1.5 - xh * y * y)
    return y


def _make_kernel(B, V4, D):
    assert B % (NW * GRP) == 0
    b_per_w = B // NW
    n_chunks = b_per_w // GRP
    assert n_chunks % 2 == 0
    n_pairs = n_chunks // 2
    out_rows_w = b_per_w * D // 128
    chunk_out = GRP * D // 128

    mesh = plsc.VectorSubcoreMesh(
        core_axis_name="c", subcore_axis_name="s", num_cores=NC, num_subcores=NS
    )

    @functools.partial(
        pl.kernel,
        out_type=jax.ShapeDtypeStruct((B * D // 128, 128), jnp.float32),
        mesh=mesh,
        scratch_types=[
            pltpu.VMEM((b_per_w,), jnp.int32),
            pltpu.VMEM((b_per_w,), jnp.int32),
            pltpu.VMEM((GRP, 128), jnp.float32),
            pltpu.VMEM((GRP, 128), jnp.float32),
            pltpu.VMEM((chunk_out, 128), jnp.float32),
            pltpu.VMEM((chunk_out, 128), jnp.float32),
            pltpu.SemaphoreType.DMA,
            pltpu.SemaphoreType.DMA,
            pltpu.SemaphoreType.DMA,
            pltpu.SemaphoreType.DMA,
        ],
        compiler_params=pltpu.CompilerParams(needs_layout_passes=False),
    )
    def fused_kernel(idx4_hbm, off_hbm, table_hbm, out_hbm,
                     idx4_v, off_v, rowsA, rowsB, outA, outB,
                     gsemA, gsemB, wsemA, wsemB):
        wid = lax.axis_index("s") * NC + lax.axis_index("c")
        base = wid * b_per_w
        obase = wid * out_rows_w

        pltpu.sync_copy(idx4_hbm.at[pl.ds(base, b_per_w)], idx4_v)
        pltpu.sync_copy(off_hbm.at[pl.ds(base, b_per_w)], off_v)

        def fire_gather(c, rows_buf, sem):
            s = pl.multiple_of(c * GRP, GRP)
            return pltpu.async_copy(
                table_hbm.at[idx4_v.at[pl.ds(s, GRP)]], rows_buf, sem
            )

        def wait_gather(rows_buf, sem):
            pltpu.make_async_copy(
                table_hbm.at[pl.ds(0, GRP)], rows_buf, sem
            ).wait()

        def fire_write(c, out_buf, sem):
            s = pl.multiple_of(obase + c * chunk_out, chunk_out)
            return pltpu.async_copy(
                out_buf, out_hbm.at[pl.ds(s, chunk_out)], sem
            )

        def wait_write(out_buf, sem):
            pltpu.make_async_copy(
                out_buf, out_hbm.at[pl.ds(0, chunk_out)], sem
            ).wait()

        iota = lax.iota(jnp.int32, 16)

        def compute(rows_buf, out_buf, cpos):
            cb = cpos * GRP
            for g in range(GRP // ROWS_PER_GROUP):
                r = iota + g * ROWS_PER_GROUP
                off = off_v[pl.ds(cb + g * ROWS_PER_GROUP, ROWS_PER_GROUP)]
                vs = []
                acc = jnp.zeros((16,), jnp.float32)
                for j in range(D):
                    v = plsc.load_gather(rows_buf, [r, off + j])
                    vs.append(v)
                    acc = acc + v * v
                inv = _rsqrt(jnp.maximum(acc, 1e-24))
                flat0 = r * D
                for j in range(D):
                    flat = flat0 + j
                    plsc.store_scatter(
                        out_buf,
                        [lax.shift_right_logical(flat, 7),
                         lax.bitwise_and(flat, 127)],
                        vs[j] * inv,
                    )

        fire_gather(0, rowsA, gsemA)

        def body(k, carry):
            a = 2 * k
            b = a + 1
            wait_gather(rowsA, gsemA)
            dB = fire_gather(b, rowsB, gsemB)

            @pl.when(k > 0)
            def _():
                wait_write(outA, wsemA)

            compute(rowsA, outA, a)
            fire_write(a, outA, wsemA)

            @pl.when(k < n_pairs - 1)
            def _():
                fire_gather(a + 2, rowsA, gsemA)

            dB.wait()

            @pl.when(k > 0)
            def _():
                wait_write(outB, wsemB)

            compute(rowsB, outB, b)
            fire_write(b, outB, wsemB)
            return carry

        lax.fori_loop(0, n_pairs, body, 0)
        wait_write(outA, wsemA)
        wait_write(outB, wsemB)

    return fused_kernel


def kernel(input, W):
    batch, fields = input.shape
    V, Dw = W.shape
    B = batch * fields
    idx = input.reshape(B).astype(jnp.int32)
    idx4 = lax.shift_right_logical(idx, 2)
    off = lax.shift_left(jnp.bitwise_and(idx, 3), 5)
    table = W.reshape(V // 4, 128)
    out = _make_kernel(B, V // 4, Dw)(idx4, off, table)
    return out.reshape(batch, fields, Dw)


# speedup vs baseline: 1.2954x; 1.1307x over previous
import functools

import jax
import jax.numpy as jnp
from jax import lax
from jax.experimental import pallas as pl
from jax.experimental.pallas import tpu as pltpu
from jax.experimental.pallas import tpu_sc as plsc

NC = 2
NS = 16
NW = NC * NS

V = 1000000
D = 32
VCHUNK = 256
N_FULL = V // VCHUNK
ROWS_PER_CHUNK = VCHUNK * D // 128
VR = V * D // 128 + 16
GRP = 128
NBUF = 4


def _rsqrt(x):
    i = plsc.bitcast(x, jnp.int32)
    i = 0x5F3759DF - lax.shift_right_logical(i, 1)
    y = plsc.bitcast(i, jnp.float32)
    xh = x * 0.5
    for _ in range(3):
        y = y * (1.5 - xh * y * y)
    return y


def _make_transpose():
    mesh = plsc.VectorSubcoreMesh(
        core_axis_name="c", subcore_axis_name="s", num_cores=NC, num_subcores=NS
    )
    n_iter = (N_FULL + NW - 1) // NW

    @functools.partial(
        pl.kernel,
        out_type=jax.ShapeDtypeStruct((VR, 128), jnp.float32),
        mesh=mesh,
        scratch_types=[
            pltpu.VMEM((D, VCHUNK), jnp.float32),
            pltpu.VMEM((D, VCHUNK), jnp.float32),
            pltpu.VMEM((ROWS_PER_CHUNK, 128), jnp.float32),
            pltpu.VMEM((ROWS_PER_CHUNK, 128), jnp.float32),
            pltpu.SemaphoreType.DMA,
            pltpu.SemaphoreType.DMA,
            pltpu.SemaphoreType.DMA,
            pltpu.SemaphoreType.DMA,
        ],
        compiler_params=pltpu.CompilerParams(needs_layout_passes=False),
    )
    def transpose_kernel(wt_hbm, out_hbm, inA, inB, trA, trB,
                         isemA, isemB, osemA, osemB):
        wid = lax.axis_index("s") * NC + lax.axis_index("c")
        iota = lax.iota(jnp.int32, 16)

        def fire_in(c, buf, sem):
            v0 = pl.multiple_of(c * VCHUNK, VCHUNK)
            return pltpu.async_copy(
                wt_hbm.at[:, pl.ds(v0, VCHUNK)], buf, sem)

        def wait_in(buf, sem):
            pltpu.make_async_copy(
                wt_hbm.at[:, pl.ds(0, VCHUNK)], buf, sem).wait()

        def fire_out(c, buf, sem):
            r0 = pl.multiple_of(c * ROWS_PER_CHUNK, ROWS_PER_CHUNK)
            return pltpu.async_copy(
                buf, out_hbm.at[pl.ds(r0, ROWS_PER_CHUNK)], sem)

        def wait_out(buf, sem):
            pltpu.make_async_copy(
                buf, out_hbm.at[pl.ds(0, ROWS_PER_CHUNK)], sem).wait()

        def transpose_chunk(in_buf, tr_buf):
            def vv_body(q, carry):
                rows2 = [jnp.full((16,), q * 2 + s, jnp.int32)
                         for s in range(2)]
                for k in range(8):
                    vsplat = jnp.full((16,), q * 8 + k, jnp.int32)
                    for d0 in (0, 16):
                        fs = k * D + d0
                        x = plsc.load_gather(in_buf, [d0 + iota, vsplat])
                        plsc.store_scatter(
                            tr_buf,
                            [rows2[fs // 128], (fs % 128) + iota], x)
                return carry

            lax.fori_loop(0, VCHUNK // 8, vv_body, 0)

        def pair_body(p, carry):
            i = 2 * p
            c = i * NW + wid
            c2 = (i + 1) * NW + wid

            @pl.when(c < N_FULL)
            def _():
                wait_in(inA, isemA)

                @pl.when(p > 0)
                def _():
                    wait_out(trA, osemA)

                transpose_chunk(inA, trA)

                @pl.when(c2 + NW < N_FULL)
                def _():
                    fire_in(c2 + NW, inA, isemA)

                fire_out(c, trA, osemA)

            @pl.when(c2 < N_FULL)
            def _():
                wait_in(inB, isemB)

                @pl.when(p > 0)
                def _():
                    wait_out(trB, osemB)

                transpose_chunk(inB, trB)

                @pl.when(c2 + 2 * NW < N_FULL)
                def _():
                    fire_in(c2 + 2 * NW, inB, isemB)

                fire_out(c2, trB, osemB)
            return carry

        @pl.when(wid < N_FULL)
        def _():
            fire_in(wid, inA, isemA)

        @pl.when(wid + NW < N_FULL)
        def _():
            fire_in(wid + NW, inB, isemB)

        n_pairs = (n_iter + 1) // 2
        lax.fori_loop(0, n_pairs, pair_body, 0)

        @pl.when(wid < N_FULL)
        def _():
            wait_out(trA, osemA)

        @pl.when(wid + NW < N_FULL)
        def _():
            wait_out(trB, osemB)

        @pl.when(wid == 0)
        def _():
            v0 = pl.multiple_of(N_FULL * VCHUNK, 128)
            pltpu.async_copy(
                wt_hbm.at[:, pl.ds(v0, 128)],
                inA.at[:, pl.ds(0, 128)], isemA).wait()

            def tail_body(q, carry):
                rows2 = [jnp.full((16,), q * 2 + s, jnp.int32)
                         for s in range(2)]
                for k in range(8):
                    vsplat = jnp.full((16,), q * 8 + k, jnp.int32)
                    for d0 in (0, 16):
                        fs = k * D + d0
                        x = plsc.load_gather(inA, [d0 + iota, vsplat])
                        plsc.store_scatter(
                            trA, [rows2[fs // 128], (fs % 128) + iota], x)
                return carry

            lax.fori_loop(0, 16, tail_body, 0)
            r0 = pl.multiple_of(N_FULL * ROWS_PER_CHUNK, 32)
            pltpu.async_copy(
                trA.at[pl.ds(0, 32)],
                out_hbm.at[pl.ds(r0, 32)], osemA).wait()

    return transpose_kernel


def _make_gather(B, BATCH, FIELDS):
    b_per_w = BATCH // NW
    n_per_w = b_per_w * FIELDS
    assert b_per_w * D // 16 % 32 == 0

    mesh = plsc.VectorSubcoreMesh(
        core_axis_name="c", subcore_axis_name="s", num_cores=NC, num_subcores=NS
    )

    @functools.partial(
        pl.kernel,
        out_type=jax.ShapeDtypeStruct((FIELDS, D, BATCH), jnp.float32),
        mesh=mesh,
        scratch_types=[
            pltpu.VMEM((n_per_w,), jnp.int32),
            pltpu.VMEM((n_per_w,), jnp.int32),
            pltpu.VMEM((NBUF * GRP, 128), jnp.float32),
            pltpu.VMEM((D, b_per_w), jnp.float32),
            pltpu.SemaphoreType.DMA((NBUF,)),
            pltpu.SemaphoreType.DMA,
        ],
        compiler_params=pltpu.CompilerParams(needs_layout_passes=False),
    )
    def gather_kernel(idx4_hbm, off_hbm, table_hbm, out_hbm,
                      idx4_v, off_v, rows, out_v, gsems, wsem):
        wid = lax.axis_index("s") * NC + lax.axis_index("c")
        base = wid * n_per_w
        b0 = wid * b_per_w
        iota = lax.iota(jnp.int32, 16)
        n_chunks = FIELDS * (b_per_w // GRP)

        pltpu.sync_copy(idx4_hbm.at[pl.ds(base, n_per_w)], idx4_v)
        pltpu.sync_copy(off_hbm.at[pl.ds(base, n_per_w)], off_v)

        def fire_gather(c, slot):
            s = pl.multiple_of(c * GRP, GRP)
            return pltpu.async_copy(
                table_hbm.at[idx4_v.at[pl.ds(s, GRP)]],
                rows.at[pl.ds(slot * GRP, GRP)], gsems.at[slot])

        def wait_gather(slot):
            pltpu.make_async_copy(
                table_hbm.at[pl.ds(0, GRP)],
                rows.at[pl.ds(0, GRP)], gsems.at[slot]).wait()

        for j in range(NBUF):
            fire_gather(j, j)

        cpf = b_per_w // GRP

        def body(c, carry):
            slot = lax.rem(c, NBUF)
            j = lax.rem(c, cpf)
            f = lax.div(c, cpf)

            @pl.when(jnp.logical_and(j == 0, f > 0))
            def _():
                pltpu.make_async_copy(
                    out_v, out_hbm.at[0, :, pl.ds(0, b_per_w)], wsem).wait()

            wait_gather(slot)
            rbase = slot * GRP
            for g in range(GRP // 16):
                off = off_v[pl.ds(c * GRP + g * 16, 16)]
                r = iota + rbase + g * 16
                vs = []
                acc = jnp.zeros((16,), jnp.float32)
                for jj in range(D):
                    v = plsc.load_gather(rows, [r, off + jj])
                    vs.append(v)
                    acc = acc + v * v
                inv = _rsqrt(jnp.maximum(acc, 1e-24))
                col = j * GRP + g * 16
                for jj in range(D):
                    out_v[jj, pl.ds(col, 16)] = vs[jj] * inv

            @pl.when(c + NBUF < n_chunks)
            def _():
                fire_gather(c + NBUF, slot)

            @pl.when(j == cpf - 1)
            def _():
                pltpu.async_copy(
                    out_v, out_hbm.at[f, :, pl.ds(b0, b_per_w)], wsem)
            return carry

        lax.fori_loop(0, n_chunks, body, 0)
        pltpu.make_async_copy(
            out_v, out_hbm.at[0, :, pl.ds(0, b_per_w)], wsem).wait()

    return gather_kernel


def kernel(input, W):
    batch, fields = input.shape
    Vw, Dw = W.shape
    B = batch * fields
    idx = input.reshape(NW, batch // NW, fields).transpose(0, 2, 1).reshape(B)
    idx = idx.astype(jnp.int32)
    idx4 = lax.shift_right_logical(idx, 2)
    off = lax.shift_left(jnp.bitwise_and(idx, 3), 5)
    wt = W.T
    table = _make_transpose()(wt)
    out = _make_gather(B, batch, fields)(idx4, off, table)
    return out.transpose(2, 0, 1)
